# initial kernel scaffold (unmeasured)
import jax
import jax.numpy as jnp
from jax import lax
from jax.experimental import pallas as pl
from jax.experimental.pallas import tpu as pltpu

N_DEV = 8


def kernel(x, w_mat):
    m_per, k = x.shape
    _, n_per = w_mat.shape

    def body(x_ref, w_ref, out_ref, xg_ref, amax_ref,
             x_send_sems, x_recv_sems, a_send_sems, a_recv_sems):
        me = lax.axis_index("i")
        left = lax.rem(me + N_DEV - 1, N_DEV)
        right = lax.rem(me + 1, N_DEV)

        barrier_sem = pltpu.get_barrier_semaphore()
        for nbr in (left, right):
            pl.semaphore_signal(
                barrier_sem, inc=1,
                device_id=(nbr,), device_id_type=pl.DeviceIdType.MESH,
            )
        pl.semaphore_wait(barrier_sem, 2)

        amax = jnp.float32(0.0)

        def gemm_block(src_block, origin):
            blk = jnp.dot(src_block, w_ref[...],
                          preferred_element_type=jnp.float32)
            out_ref[pl.ds(origin * m_per, m_per), :] = blk
            return jnp.max(jnp.abs(blk))

        send_rdmas = []
        for h in range(N_DEV - 1):
            origin_s = lax.rem(me - h + N_DEV, N_DEV)
            origin_r = lax.rem(me - 1 - h + N_DEV, N_DEV)
            src = x_ref if h == 0 else xg_ref.at[origin_s]
            send = pltpu.make_async_remote_copy(
                src_ref=src,
                dst_ref=xg_ref.at[origin_s],
                send_sem=x_send_sems.at[h],
                recv_sem=x_recv_sems.at[origin_s],
                device_id=(right,),
                device_id_type=pl.DeviceIdType.MESH,
            )
            send.start()
            send_rdmas.append(send)

            if h == 0:
                amax = jnp.maximum(amax, gemm_block(x_ref[...], me))
            else:
                amax = jnp.maximum(
                    amax, gemm_block(xg_ref[origin_s], origin_s))

            recv = pltpu.make_async_remote_copy(
                src_ref=xg_ref.at[origin_r],
                dst_ref=xg_ref.at[origin_r],
                send_sem=x_send_sems.at[h],
                recv_sem=x_recv_sems.at[origin_r],
                device_id=(left,),
                device_id_type=pl.DeviceIdType.MESH,
            )
            recv.wait_recv()

        last_origin = lax.rem(me + 1, N_DEV)
        amax = jnp.maximum(amax, gemm_block(xg_ref[last_origin], last_origin))

        for send in send_rdmas:
            send.wait_send()

        amax_ref[pl.ds(me, 1), :] = jnp.full((1, 128), amax, jnp.float32)
        a_rdmas = []
        for off in range(1, N_DEV):
            peer = lax.rem(me + off, N_DEV)
            rdma = pltpu.make_async_remote_copy(
                src_ref=amax_ref.at[pl.ds(me, 1)],
                dst_ref=amax_ref.at[pl.ds(me, 1)],
                send_sem=a_send_sems.at[off],
                recv_sem=a_recv_sems.at[me],
                device_id=(peer,),
                device_id_type=pl.DeviceIdType.MESH,
            )
            rdma.start()
            a_rdmas.append(rdma)
        for off in range(1, N_DEV):
            origin = lax.rem(me + off, N_DEV)
            recv = pltpu.make_async_remote_copy(
                src_ref=amax_ref.at[pl.ds(origin, 1)],
                dst_ref=amax_ref.at[pl.ds(origin, 1)],
                send_sem=a_send_sems.at[off],
                recv_sem=a_recv_sems.at[origin],
                device_id=(left,),
                device_id_type=pl.DeviceIdType.MESH,
            )
            recv.wait_recv()
        for rdma in a_rdmas:
            rdma.wait_send()

        gmax = jnp.max(amax_ref[...])
        scale = gmax / 127.0

        for j in range(N_DEV):
            rows = pl.ds(j * m_per, m_per)
            y = out_ref[rows, :]
            q = jnp.clip(jnp.round(y / scale), -127.0, 127.0)
            out_ref[rows, :] = q * scale

    return pl.pallas_call(
        body,
        out_shape=jax.ShapeDtypeStruct((N_DEV * m_per, n_per), jnp.float32),
        in_specs=[
            pl.BlockSpec(memory_space=pltpu.VMEM),
            pl.BlockSpec(memory_space=pltpu.VMEM),
        ],
        out_specs=pl.BlockSpec(memory_space=pltpu.VMEM),
        scratch_shapes=[
            pltpu.VMEM((N_DEV, m_per, k), jnp.float32),
            pltpu.VMEM((N_DEV, 128), jnp.float32),
            pltpu.SemaphoreType.DMA((N_DEV - 1,)),
            pltpu.SemaphoreType.DMA((N_DEV,)),
            pltpu.SemaphoreType.DMA((N_DEV,)),
            pltpu.SemaphoreType.DMA((N_DEV,)),
        ],
        compiler_params=pltpu.CompilerParams(collective_id=0),
    )(x, w_mat)


# baseline (device time: 694329 ns/iter reference)
import jax
import jax.numpy as jnp
from jax import lax
from jax.experimental import pallas as pl
from jax.experimental.pallas import tpu as pltpu

N_DEV = 8


def kernel(x, w_mat):
    m_per, k = x.shape
    _, n_per = w_mat.shape

    def body(x_ref, w_ref, out_ref, xg_ref, stage_ref, amax_ref,
             x_send_sems, x_recv_sems, copy_sems, a_send_sems, a_recv_sems):
        me = lax.axis_index("i")
        left = lax.rem(me + N_DEV - 1, N_DEV)
        right = lax.rem(me + 1, N_DEV)

        barrier_sem = pltpu.get_barrier_semaphore()
        for nbr in (left, right):
            pl.semaphore_signal(
                barrier_sem, inc=1,
                device_id=(nbr,), device_id_type=pl.DeviceIdType.MESH,
            )
        pl.semaphore_wait(barrier_sem, 2)

        amax = jnp.float32(0.0)

        def gemm_block(src_block, origin, acc):
            blk = jnp.dot(src_block, w_ref[...],
                          preferred_element_type=jnp.float32)
            out_ref[pl.ds(origin * m_per, m_per), :] = blk
            return jnp.maximum(acc, jnp.max(jnp.abs(blk)))

        send_rdmas = []
        copies = [None, None]
        for h in range(N_DEV - 1):
            origin_s = lax.rem(me - h + N_DEV, N_DEV)
            origin_r = lax.rem(me - 1 - h + N_DEV, N_DEV)
            src = x_ref if h == 0 else xg_ref.at[origin_s]
            send = pltpu.make_async_remote_copy(
                src_ref=src,
                dst_ref=xg_ref.at[origin_s],
                send_sem=x_send_sems.at[h],
                recv_sem=x_recv_sems.at[origin_s],
                device_id=(right,),
                device_id_type=pl.DeviceIdType.MESH,
            )
            send.start()
            send_rdmas.append(send)

            if h == 0:
                amax = gemm_block(x_ref[...], me, amax)
            else:
                copies[h % 2].wait()
                amax = gemm_block(stage_ref[h % 2], origin_s, amax)

            recv = pltpu.make_async_remote_copy(
                src_ref=xg_ref.at[origin_r],
                dst_ref=xg_ref.at[origin_r],
                send_sem=x_send_sems.at[h],
                recv_sem=x_recv_sems.at[origin_r],
                device_id=(left,),
                device_id_type=pl.DeviceIdType.MESH,
            )
            recv.wait_recv()

            cp = pltpu.make_async_copy(
                xg_ref.at[origin_r],
                stage_ref.at[(h + 1) % 2],
                copy_sems.at[(h + 1) % 2],
            )
            cp.start()
            copies[(h + 1) % 2] = cp

        last_origin = lax.rem(me + 1, N_DEV)
        copies[(N_DEV - 1) % 2].wait()
        amax = gemm_block(stage_ref[(N_DEV - 1) % 2], last_origin, amax)

        for send in send_rdmas:
            send.wait_send()

        amax_ref[pl.ds(me, 1), :] = jnp.full((1, 128), amax, jnp.float32)
        a_rdmas = []
        for off in range(1, N_DEV):
            peer = lax.rem(me + off, N_DEV)
            rdma = pltpu.make_async_remote_copy(
                src_ref=amax_ref.at[pl.ds(me, 1)],
                dst_ref=amax_ref.at[pl.ds(me, 1)],
                send_sem=a_send_sems.at[off],
                recv_sem=a_recv_sems.at[me],
                device_id=(peer,),
                device_id_type=pl.DeviceIdType.MESH,
            )
            rdma.start()
            a_rdmas.append(rdma)
        for off in range(1, N_DEV):
            origin = lax.rem(me + off, N_DEV)
            recv = pltpu.make_async_remote_copy(
                src_ref=amax_ref.at[pl.ds(origin, 1)],
                dst_ref=amax_ref.at[pl.ds(origin, 1)],
                send_sem=a_send_sems.at[off],
                recv_sem=a_recv_sems.at[origin],
                device_id=(left,),
                device_id_type=pl.DeviceIdType.MESH,
            )
            recv.wait_recv()
        for rdma in a_rdmas:
            rdma.wait_send()

        gmax = jnp.max(amax_ref[...])
        scale = gmax / 127.0

        for j in range(N_DEV):
            rows = pl.ds(j * m_per, m_per)
            y = out_ref[rows, :]
            q = jnp.clip(jnp.round(y / scale), -127.0, 127.0)
            out_ref[rows, :] = q * scale

    out, _xg = pl.pallas_call(
        body,
        out_shape=[
            jax.ShapeDtypeStruct((N_DEV * m_per, n_per), jnp.float32),
            jax.ShapeDtypeStruct((N_DEV, m_per, k), jnp.float32),
        ],
        in_specs=[
            pl.BlockSpec(memory_space=pltpu.VMEM),
            pl.BlockSpec(memory_space=pltpu.VMEM),
        ],
        out_specs=[
            pl.BlockSpec(memory_space=pltpu.VMEM),
            pl.BlockSpec(memory_space=pl.ANY),
        ],
        scratch_shapes=[
            pltpu.VMEM((2, m_per, k), jnp.float32),
            pltpu.VMEM((N_DEV, 128), jnp.float32),
            pltpu.SemaphoreType.DMA((N_DEV - 1,)),
            pltpu.SemaphoreType.DMA((N_DEV,)),
            pltpu.SemaphoreType.DMA((2,)),
            pltpu.SemaphoreType.DMA((N_DEV,)),
            pltpu.SemaphoreType.DMA((N_DEV,)),
        ],
        compiler_params=pltpu.CompilerParams(
            collective_id=0, vmem_limit_bytes=63 * 1024 * 1024),
    )(x, w_mat)
    return out


# device time: 370656 ns/iter; 1.8732x vs baseline; 1.8732x over previous
import jax
import jax.numpy as jnp
from jax import lax
from jax.experimental import pallas as pl
from jax.experimental.pallas import tpu as pltpu

N_DEV = 8
N_HOPS = 4


def kernel(x, w_mat):
    m_per, k = x.shape
    _, n_per = w_mat.shape
    half = m_per // 2

    def body(x_ref, w_ref, out_ref, xg_ref, stage_ref, amax_ref,
             cw_send_sems, cw_recv_sems, ccw_send_sems, ccw_recv_sems,
             copy_sems, a_send_sems, a_recv_sems):
        me = lax.axis_index("i")
        left = lax.rem(me + N_DEV - 1, N_DEV)
        right = lax.rem(me + 1, N_DEV)

        barrier_sem = pltpu.get_barrier_semaphore()
        for nbr in (left, right):
            pl.semaphore_signal(
                barrier_sem, inc=1,
                device_id=(nbr,), device_id_type=pl.DeviceIdType.MESH,
            )
        pl.semaphore_wait(barrier_sem, 2)

        def slot(origin, h):
            if h < N_HOPS - 1:
                return xg_ref.at[origin]
            return None

        def mk_cw_send(h):
            origin = lax.rem(me - h + N_DEV, N_DEV)
            if h == 0:
                src = x_ref
                dst = xg_ref.at[origin]
            elif h < N_HOPS - 1:
                src = xg_ref.at[origin]
                dst = xg_ref.at[origin]
            else:
                src = xg_ref.at[origin, pl.ds(0, half), :]
                dst = xg_ref.at[origin, pl.ds(0, half), :]
            return pltpu.make_async_remote_copy(
                src_ref=src, dst_ref=dst,
                send_sem=cw_send_sems.at[h], recv_sem=cw_recv_sems.at[origin],
                device_id=(right,), device_id_type=pl.DeviceIdType.MESH)

        def mk_ccw_send(h):
            origin = lax.rem(me + h, N_DEV)
            if h == 0:
                src = x_ref
                dst = xg_ref.at[origin]
            elif h < N_HOPS - 1:
                src = xg_ref.at[origin]
                dst = xg_ref.at[origin]
            else:
                src = xg_ref.at[origin, pl.ds(half, half), :]
                dst = xg_ref.at[origin, pl.ds(half, half), :]
            return pltpu.make_async_remote_copy(
                src_ref=src, dst_ref=dst,
                send_sem=ccw_send_sems.at[h], recv_sem=ccw_recv_sems.at[origin],
                device_id=(left,), device_id_type=pl.DeviceIdType.MESH)

        def wait_recv(origin, sems, is_half, half_off):
            if is_half:
                dst = xg_ref.at[origin, pl.ds(half_off, half), :]
            else:
                dst = xg_ref.at[origin]
            rdma = pltpu.make_async_remote_copy(
                src_ref=dst, dst_ref=dst,
                send_sem=cw_send_sems.at[0], recv_sem=sems.at[origin],
                device_id=(left,), device_id_type=pl.DeviceIdType.MESH)
            rdma.wait_recv()

        amax = jnp.float32(0.0)

        def gemm_block(src_block, origin, acc):
            blk = jnp.dot(src_block, w_ref[...],
                          preferred_element_type=jnp.float32)
            out_ref[pl.ds(origin * m_per, m_per), :] = blk
            return jnp.maximum(acc, jnp.max(jnp.abs(blk)))

        def gemm_half(src_block, origin, i, acc):
            blk = jnp.dot(src_block, w_ref[...],
                          preferred_element_type=jnp.float32)
            out_ref[pl.ds(origin * m_per + i * half, half), :] = blk
            return jnp.maximum(acc, jnp.max(jnp.abs(blk)))

        def stage_gemm(origin, acc):
            cps = []
            for i in range(2):
                cp = pltpu.make_async_copy(
                    xg_ref.at[origin, pl.ds(i * half, half), :],
                    stage_ref.at[i], copy_sems.at[i])
                cp.start()
                cps.append(cp)
            for i in range(2):
                cps[i].wait()
                acc = gemm_half(stage_ref[i], origin, i, acc)
            return acc

        sends = [mk_cw_send(0), mk_ccw_send(0)]
        sends[0].start()
        sends[1].start()

        amax = gemm_block(x_ref[...], me, amax)

        for h in range(N_HOPS):
            o_cw = lax.rem(me - 1 - h + N_DEV, N_DEV)
            o_ccw = lax.rem(me + 1 + h, N_DEV)
            is_last = h == N_HOPS - 1
            wait_recv(o_cw, cw_recv_sems, is_last, 0)
            if not is_last:
                s = mk_cw_send(h + 1)
                s.start()
                sends.append(s)
            wait_recv(o_ccw, ccw_recv_sems, is_last, half)
            if not is_last:
                s = mk_ccw_send(h + 1)
                s.start()
                sends.append(s)
            amax = stage_gemm(o_cw, amax)
            if not is_last:
                amax = stage_gemm(o_ccw, amax)

        for s in sends:
            s.wait_send()

        amax_ref[pl.ds(me, 1), :] = jnp.full((1, 128), amax, jnp.float32)
        a_rdmas = []
        for off in range(1, N_DEV):
            peer = lax.rem(me + off, N_DEV)
            rdma = pltpu.make_async_remote_copy(
                src_ref=amax_ref.at[pl.ds(me, 1)],
                dst_ref=amax_ref.at[pl.ds(me, 1)],
                send_sem=a_send_sems.at[off],
                recv_sem=a_recv_sems.at[me],
                device_id=(peer,), device_id_type=pl.DeviceIdType.MESH)
            rdma.start()
            a_rdmas.append(rdma)
        for off in range(1, N_DEV):
            origin = lax.rem(me + off, N_DEV)
            recv = pltpu.make_async_remote_copy(
                src_ref=amax_ref.at[pl.ds(origin, 1)],
                dst_ref=amax_ref.at[pl.ds(origin, 1)],
                send_sem=a_send_sems.at[off],
                recv_sem=a_recv_sems.at[origin],
                device_id=(left,), device_id_type=pl.DeviceIdType.MESH)
            recv.wait_recv()
        for rdma in a_rdmas:
            rdma.wait_send()

        gmax = jnp.max(amax_ref[...])
        scale = gmax / 127.0

        for j in range(N_DEV):
            rows = pl.ds(j * m_per, m_per)
            y = out_ref[rows, :]
            q = jnp.clip(jnp.round(y / scale), -127.0, 127.0)
            out_ref[rows, :] = q * scale

    out, _xg = pl.pallas_call(
        body,
        out_shape=[
            jax.ShapeDtypeStruct((N_DEV * m_per, n_per), jnp.float32),
            jax.ShapeDtypeStruct((N_DEV, m_per, k), jnp.float32),
        ],
        in_specs=[
            pl.BlockSpec(memory_space=pltpu.VMEM),
            pl.BlockSpec(memory_space=pltpu.VMEM),
        ],
        out_specs=[
            pl.BlockSpec(memory_space=pltpu.VMEM),
            pl.BlockSpec(memory_space=pl.ANY),
        ],
        scratch_shapes=[
            pltpu.VMEM((2, m_per // 2, k), jnp.float32),
            pltpu.VMEM((N_DEV, 128), jnp.float32),
            pltpu.SemaphoreType.DMA((N_HOPS,)),
            pltpu.SemaphoreType.DMA((N_DEV,)),
            pltpu.SemaphoreType.DMA((N_HOPS,)),
            pltpu.SemaphoreType.DMA((N_DEV,)),
            pltpu.SemaphoreType.DMA((2,)),
            pltpu.SemaphoreType.DMA((N_DEV,)),
            pltpu.SemaphoreType.DMA((N_DEV,)),
        ],
        compiler_params=pltpu.CompilerParams(
            collective_id=0, vmem_limit_bytes=63 * 1024 * 1024),
    )(x, w_mat)
    return out


# device time: 322409 ns/iter; 2.1536x vs baseline; 1.1496x over previous
import jax
import jax.numpy as jnp
from jax import lax
from jax.experimental import pallas as pl
from jax.experimental.pallas import tpu as pltpu

N_DEV = 8
RING = [0, 4, 7, 3, 2, 6, 5, 1]
IDX = [0, 7, 4, 3, 1, 6, 5, 2]


def kernel(x, w_mat):
    m_per, k = x.shape
    _, n_per = w_mat.shape
    half = m_per // 2

    def body(x_ref, w_ref, out_ref, xg_ref, stage_ref, amax_ref,
             cw_send_sems, ccw_send_sems, ch_send_sems, recv_sems,
             copy_sems, a_send_sems, a_recv_sems):
        me_pos = lax.axis_index("i")

        def lut(i, table):
            v = jnp.int32(table[0])
            for j in range(1, N_DEV):
                v = jnp.where(i == j, jnp.int32(table[j]), v)
            return v

        def m8(v):
            return lax.rem(v + 2 * N_DEV, N_DEV)

        idx = lut(me_pos, IDX)
        par = lax.rem(idx, 2)
        is_even = par == 0
        is_odd = par == 1

        left_pos = lut(m8(idx - 1), RING)
        right_pos = lut(m8(idx + 1), RING)
        p_idx = m8(idx + 3 - 6 * par)
        p_pos = lut(p_idx, RING)

        barrier_sem = pltpu.get_barrier_semaphore()
        for nbr in (left_pos, right_pos, p_pos):
            pl.semaphore_signal(
                barrier_sem, inc=1,
                device_id=(nbr,), device_id_type=pl.DeviceIdType.MESH,
            )
        pl.semaphore_wait(barrier_sem, 3)

        def mk_send(src, o_idx, to_pos, sem):
            return pltpu.make_async_remote_copy(
                src_ref=src, dst_ref=xg_ref.at[o_idx],
                send_sem=sem, recv_sem=recv_sems.at[o_idx],
                device_id=(to_pos,), device_id_type=pl.DeviceIdType.MESH)

        def wait_origin(o_idx):
            rdma = pltpu.make_async_remote_copy(
                src_ref=xg_ref.at[o_idx], dst_ref=xg_ref.at[o_idx],
                send_sem=cw_send_sems.at[0], recv_sem=recv_sems.at[o_idx],
                device_id=(left_pos,), device_id_type=pl.DeviceIdType.MESH)
            rdma.wait_recv()

        amax = jnp.float32(0.0)

        def gemm_half(src_block, row_base, i, acc):
            blk = jnp.dot(src_block, w_ref[...],
                          preferred_element_type=jnp.float32)
            out_ref[pl.ds(row_base + i * half, half), :] = blk
            return jnp.maximum(acc, jnp.max(jnp.abs(blk)))

        def stage_gemm(o_idx, acc):
            row_base = lut(o_idx, RING) * m_per
            cps = []
            for i in range(2):
                cp = pltpu.make_async_copy(
                    xg_ref.at[o_idx, pl.ds(i * half, half), :],
                    stage_ref.at[i], copy_sems.at[i])
                cp.start()
                cps.append(cp)
            for i in range(2):
                cps[i].wait()
                acc = gemm_half(stage_ref[i], row_base, i, acc)
            return acc

        o_cw0, o_ccw0 = m8(idx - 1), m8(idx + 1)
        o_cw1, o_ccw1 = m8(idx - 2), m8(idx + 2)
        o_c1 = m8(idx + 4)
        c1_src = m8(idx + 2 * par - 1)
        o_d3 = m8(idx - 3 + 6 * par)

        cw0 = mk_send(x_ref, idx, right_pos, cw_send_sems.at[0])
        ccw0 = mk_send(x_ref, idx, left_pos, ccw_send_sems.at[0])
        c0 = mk_send(x_ref, idx, p_pos, ch_send_sems.at[0])
        cw0.start()
        ccw0.start()
        c0.start()

        row_base = me_pos * m_per
        amax = gemm_half(x_ref[pl.ds(0, half), :], row_base, 0, amax)
        amax = gemm_half(x_ref[pl.ds(half, half), :], row_base, 1, amax)

        wait_origin(o_cw0)
        cw1 = mk_send(xg_ref.at[o_cw0], o_cw0, right_pos, cw_send_sems.at[1])
        cw1.start()
        wait_origin(o_ccw0)
        ccw1 = mk_send(xg_ref.at[o_ccw0], o_ccw0, left_pos,
                       ccw_send_sems.at[1])
        ccw1.start()
        c1 = mk_send(xg_ref.at[c1_src], c1_src, p_pos, ch_send_sems.at[1])
        c1.start()

        amax = stage_gemm(o_cw0, amax)
        amax = stage_gemm(o_ccw0, amax)
        wait_origin(p_idx)
        amax = stage_gemm(p_idx, amax)

        cw2 = mk_send(xg_ref.at[o_cw1], o_cw1, right_pos, cw_send_sems.at[2])
        ccw2 = mk_send(xg_ref.at[o_ccw1], o_ccw1, left_pos,
                       ccw_send_sems.at[2])
        wait_origin(o_cw1)

        @pl.when(is_odd)
        def _():
            cw2.start()

        wait_origin(o_ccw1)

        @pl.when(is_even)
        def _():
            ccw2.start()

        amax = stage_gemm(o_cw1, amax)
        amax = stage_gemm(o_ccw1, amax)

        wait_origin(o_c1)
        amax = stage_gemm(o_c1, amax)
        wait_origin(o_d3)
        amax = stage_gemm(o_d3, amax)

        for s in (cw0, cw1, ccw0, ccw1, c0, c1):
            s.wait_send()

        @pl.when(is_odd)
        def _():
            cw2.wait_send()

        @pl.when(is_even)
        def _():
            ccw2.wait_send()

        amax_ref[pl.ds(me_pos, 1), :] = jnp.full((1, 128), amax, jnp.float32)
        a_rdmas = []
        for off in range(1, N_DEV):
            peer = m8(me_pos + off)
            rdma = pltpu.make_async_remote_copy(
                src_ref=amax_ref.at[pl.ds(me_pos, 1)],
                dst_ref=amax_ref.at[pl.ds(me_pos, 1)],
                send_sem=a_send_sems.at[off],
                recv_sem=a_recv_sems.at[me_pos],
                device_id=(peer,), device_id_type=pl.DeviceIdType.MESH)
            rdma.start()
            a_rdmas.append(rdma)
        for off in range(1, N_DEV):
            origin = m8(me_pos + off)
            recv = pltpu.make_async_remote_copy(
                src_ref=amax_ref.at[pl.ds(origin, 1)],
                dst_ref=amax_ref.at[pl.ds(origin, 1)],
                send_sem=a_send_sems.at[off],
                recv_sem=a_recv_sems.at[origin],
                device_id=(left_pos,), device_id_type=pl.DeviceIdType.MESH)
            recv.wait_recv()
        for rdma in a_rdmas:
            rdma.wait_send()

        gmax = jnp.max(amax_ref[...])
        scale = gmax / 127.0

        for j in range(N_DEV):
            rows = pl.ds(j * m_per, m_per)
            y = out_ref[rows, :]
            q = jnp.clip(jnp.round(y / scale), -127.0, 127.0)
            out_ref[rows, :] = q * scale

    out, _xg = pl.pallas_call(
        body,
        out_shape=[
            jax.ShapeDtypeStruct((N_DEV * m_per, n_per), jnp.float32),
            jax.ShapeDtypeStruct((N_DEV, m_per, k), jnp.float32),
        ],
        in_specs=[
            pl.BlockSpec(memory_space=pltpu.VMEM),
            pl.BlockSpec(memory_space=pltpu.VMEM),
        ],
        out_specs=[
            pl.BlockSpec(memory_space=pltpu.VMEM),
            pl.BlockSpec(memory_space=pl.ANY),
        ],
        scratch_shapes=[
            pltpu.VMEM((2, m_per // 2, k), jnp.float32),
            pltpu.VMEM((N_DEV, 128), jnp.float32),
            pltpu.SemaphoreType.DMA((3,)),
            pltpu.SemaphoreType.DMA((3,)),
            pltpu.SemaphoreType.DMA((2,)),
            pltpu.SemaphoreType.DMA((N_DEV,)),
            pltpu.SemaphoreType.DMA((2,)),
            pltpu.SemaphoreType.DMA((N_DEV,)),
            pltpu.SemaphoreType.DMA((N_DEV,)),
        ],
        compiler_params=pltpu.CompilerParams(
            collective_id=0, vmem_limit_bytes=63 * 1024 * 1024),
    )(x, w_mat)
    return out


# device time: 320052 ns/iter; 2.1694x vs baseline; 1.0074x over previous
import jax
import jax.numpy as jnp
from jax import lax
from jax.experimental import pallas as pl
from jax.experimental.pallas import tpu as pltpu

N_DEV = 8
RING = [0, 4, 7, 3, 2, 6, 5, 1]
IDX = [0, 7, 4, 3, 1, 6, 5, 2]


def kernel(x, w_mat):
    m_per, k = x.shape
    _, n_per = w_mat.shape
    half = m_per // 2

    def body(x_ref, w_ref, out_ref, xg_ref, stage_ref, amax_ref,
             cw_send_sems, ccw_send_sems, ch_send_sems, recv_sems,
             d3b_sems, copy_sems, a_send_sems, a_recv_sems):
        me_pos = lax.axis_index("i")

        def lut(i, table):
            v = jnp.int32(table[0])
            for j in range(1, N_DEV):
                v = jnp.where(i == j, jnp.int32(table[j]), v)
            return v

        def m8(v):
            return lax.rem(v + 2 * N_DEV, N_DEV)

        idx = lut(me_pos, IDX)
        par = lax.rem(idx, 2)
        is_even = par == 0
        is_odd = par == 1

        left_pos = lut(m8(idx - 1), RING)
        right_pos = lut(m8(idx + 1), RING)
        p_idx = m8(idx + 3 - 6 * par)
        p_pos = lut(p_idx, RING)

        barrier_sem = pltpu.get_barrier_semaphore()
        for nbr in (left_pos, right_pos, p_pos):
            pl.semaphore_signal(
                barrier_sem, inc=1,
                device_id=(nbr,), device_id_type=pl.DeviceIdType.MESH,
            )
        pl.semaphore_wait(barrier_sem, 3)

        def mk_send(src, o_idx, to_pos, sem):
            return pltpu.make_async_remote_copy(
                src_ref=src, dst_ref=xg_ref.at[o_idx],
                send_sem=sem, recv_sem=recv_sems.at[o_idx],
                device_id=(to_pos,), device_id_type=pl.DeviceIdType.MESH)

        def wait_origin(o_idx):
            rdma = pltpu.make_async_remote_copy(
                src_ref=xg_ref.at[o_idx], dst_ref=xg_ref.at[o_idx],
                send_sem=cw_send_sems.at[0], recv_sem=recv_sems.at[o_idx],
                device_id=(left_pos,), device_id_type=pl.DeviceIdType.MESH)
            rdma.wait_recv()

        amax = jnp.float32(0.0)

        def gemm_half(src_block, row_base, i, acc):
            blk = jnp.dot(src_block, w_ref[...],
                          preferred_element_type=jnp.float32)
            out_ref[pl.ds(row_base + i * half, half), :] = blk
            return jnp.maximum(acc, jnp.max(jnp.abs(blk)))

        def stage_gemm(o_idx, acc):
            row_base = lut(o_idx, RING) * m_per
            cps = []
            for i in range(2):
                cp = pltpu.make_async_copy(
                    xg_ref.at[o_idx, pl.ds(i * half, half), :],
                    stage_ref.at[i], copy_sems.at[i])
                cp.start()
                cps.append(cp)
            for i in range(2):
                cps[i].wait()
                acc = gemm_half(stage_ref[i], row_base, i, acc)
            return acc

        o_cw0, o_ccw0 = m8(idx - 1), m8(idx + 1)
        o_cw1, o_ccw1 = m8(idx - 2), m8(idx + 2)
        o_c1 = m8(idx + 4)
        c1_src = m8(idx + 2 * par - 1)
        o_d3 = m8(idx - 3 + 6 * par)

        cw0 = mk_send(x_ref, idx, right_pos, cw_send_sems.at[0])
        ccw0 = mk_send(x_ref, idx, left_pos, ccw_send_sems.at[0])
        c0 = mk_send(x_ref, idx, p_pos, ch_send_sems.at[0])
        cw0.start()
        ccw0.start()
        c0.start()

        row_base = me_pos * m_per
        amax = gemm_half(x_ref[pl.ds(0, half), :], row_base, 0, amax)
        amax = gemm_half(x_ref[pl.ds(half, half), :], row_base, 1, amax)

        wait_origin(o_cw0)
        cw1 = mk_send(xg_ref.at[o_cw0], o_cw0, right_pos, cw_send_sems.at[1])
        cw1.start()
        wait_origin(o_ccw0)
        ccw1 = mk_send(xg_ref.at[o_ccw0], o_ccw0, left_pos,
                       ccw_send_sems.at[1])
        ccw1.start()
        c1 = mk_send(xg_ref.at[c1_src], c1_src, p_pos, ch_send_sems.at[1])
        c1.start()

        amax = stage_gemm(o_cw0, amax)
        amax = stage_gemm(o_ccw0, amax)
        wait_origin(p_idx)
        amax = stage_gemm(p_idx, amax)

        def mk_half(o_idx, i, to_pos, send_sem, recv_sem):
            sl = xg_ref.at[o_idx, pl.ds(i * half, half), :]
            return pltpu.make_async_remote_copy(
                src_ref=sl, dst_ref=sl, send_sem=send_sem, recv_sem=recv_sem,
                device_id=(to_pos,), device_id_type=pl.DeviceIdType.MESH)

        cw2a = mk_half(o_cw1, 0, right_pos, cw_send_sems.at[2],
                       recv_sems.at[o_cw1])
        cw2b = mk_half(o_cw1, 1, right_pos, cw_send_sems.at[3],
                       d3b_sems.at[0])
        ccw2a = mk_half(o_ccw1, 0, left_pos, ccw_send_sems.at[2],
                        recv_sems.at[o_ccw1])
        ccw2b = mk_half(o_ccw1, 1, left_pos, ccw_send_sems.at[3],
                        d3b_sems.at[0])
        wait_origin(o_cw1)

        @pl.when(is_odd)
        def _():
            cw2a.start()
            cw2b.start()

        wait_origin(o_ccw1)

        @pl.when(is_even)
        def _():
            ccw2a.start()
            ccw2b.start()

        amax = stage_gemm(o_cw1, amax)
        amax = stage_gemm(o_ccw1, amax)

        wait_origin(o_c1)
        amax = stage_gemm(o_c1, amax)

        row_d3 = lut(o_d3, RING) * m_per
        for i in range(2):
            sl = xg_ref.at[o_d3, pl.ds(i * half, half), :]
            rdma = pltpu.make_async_remote_copy(
                src_ref=sl, dst_ref=sl,
                send_sem=cw_send_sems.at[0],
                recv_sem=recv_sems.at[o_d3] if i == 0 else d3b_sems.at[0],
                device_id=(left_pos,), device_id_type=pl.DeviceIdType.MESH)
            rdma.wait_recv()
            cp = pltpu.make_async_copy(sl, stage_ref.at[i], copy_sems.at[i])
            cp.start()
            cp.wait()
            amax = gemm_half(stage_ref[i], row_d3, i, amax)

        for s in (cw0, cw1, ccw0, ccw1, c0, c1):
            s.wait_send()

        @pl.when(is_odd)
        def _():
            cw2a.wait_send()
            cw2b.wait_send()

        @pl.when(is_even)
        def _():
            ccw2a.wait_send()
            ccw2b.wait_send()

        amax_ref[pl.ds(me_pos, 1), :] = jnp.full((1, 128), amax, jnp.float32)
        a_rdmas = []
        for off in range(1, N_DEV):
            peer = m8(me_pos + off)
            rdma = pltpu.make_async_remote_copy(
                src_ref=amax_ref.at[pl.ds(me_pos, 1)],
                dst_ref=amax_ref.at[pl.ds(me_pos, 1)],
                send_sem=a_send_sems.at[off],
                recv_sem=a_recv_sems.at[me_pos],
                device_id=(peer,), device_id_type=pl.DeviceIdType.MESH)
            rdma.start()
            a_rdmas.append(rdma)
        for off in range(1, N_DEV):
            origin = m8(me_pos + off)
            recv = pltpu.make_async_remote_copy(
                src_ref=amax_ref.at[pl.ds(origin, 1)],
                dst_ref=amax_ref.at[pl.ds(origin, 1)],
                send_sem=a_send_sems.at[off],
                recv_sem=a_recv_sems.at[origin],
                device_id=(left_pos,), device_id_type=pl.DeviceIdType.MESH)
            recv.wait_recv()
        for rdma in a_rdmas:
            rdma.wait_send()

        gmax = jnp.max(amax_ref[...])
        scale = gmax / 127.0

        for j in range(N_DEV):
            rows = pl.ds(j * m_per, m_per)
            y = out_ref[rows, :]
            q = jnp.clip(jnp.round(y / scale), -127.0, 127.0)
            out_ref[rows, :] = q * scale

    out, _xg = pl.pallas_call(
        body,
        out_shape=[
            jax.ShapeDtypeStruct((N_DEV * m_per, n_per), jnp.float32),
            jax.ShapeDtypeStruct((N_DEV, m_per, k), jnp.float32),
        ],
        in_specs=[
            pl.BlockSpec(memory_space=pltpu.VMEM),
            pl.BlockSpec(memory_space=pltpu.VMEM),
        ],
        out_specs=[
            pl.BlockSpec(memory_space=pltpu.VMEM),
            pl.BlockSpec(memory_space=pl.ANY),
        ],
        scratch_shapes=[
            pltpu.VMEM((2, m_per // 2, k), jnp.float32),
            pltpu.VMEM((N_DEV, 128), jnp.float32),
            pltpu.SemaphoreType.DMA((4,)),
            pltpu.SemaphoreType.DMA((4,)),
            pltpu.SemaphoreType.DMA((2,)),
            pltpu.SemaphoreType.DMA((N_DEV,)),
            pltpu.SemaphoreType.DMA((1,)),
            pltpu.SemaphoreType.DMA((2,)),
            pltpu.SemaphoreType.DMA((N_DEV,)),
            pltpu.SemaphoreType.DMA((N_DEV,)),
        ],
        compiler_params=pltpu.CompilerParams(
            collective_id=0, vmem_limit_bytes=63 * 1024 * 1024),
    )(x, w_mat)
    return out


# device time: 289212 ns/iter; 2.4008x vs baseline; 1.1066x over previous
import jax
import jax.numpy as jnp
from jax import lax
from jax.experimental import pallas as pl
from jax.experimental.pallas import tpu as pltpu

N_DEV = 8
RING = [0, 4, 7, 3, 2, 6, 5, 1]
IDX = [0, 7, 4, 3, 1, 6, 5, 2]


def kernel(x, w_mat):
    m_per, k = x.shape
    _, n_per = w_mat.shape
    half = m_per // 2

    def body(x_ref, w_ref, out_ref, xg_ref, stage_ref, amax_ref,
             cw_send_sems, ccw_send_sems, ch_send_sems, recv_sems,
             d3b_sems, copy_sems, a_send_sems, a_recv_sems):
        me_pos = lax.axis_index("i")

        def lut(i, table):
            v = jnp.int32(table[0])
            for j in range(1, N_DEV):
                v = jnp.where(i == j, jnp.int32(table[j]), v)
            return v

        def m8(v):
            return lax.rem(v + 2 * N_DEV, N_DEV)

        idx = lut(me_pos, IDX)
        par = lax.rem(idx, 2)
        is_even = par == 0
        is_odd = par == 1

        left_pos = lut(m8(idx - 1), RING)
        right_pos = lut(m8(idx + 1), RING)
        p_idx = m8(idx + 3 - 6 * par)
        p_pos = lut(p_idx, RING)

        barrier_sem = pltpu.get_barrier_semaphore()
        for nbr in (left_pos, right_pos, p_pos):
            pl.semaphore_signal(
                barrier_sem, inc=1,
                device_id=(nbr,), device_id_type=pl.DeviceIdType.MESH,
            )
        pl.semaphore_wait(barrier_sem, 3)

        def mk_send(src, o_idx, to_pos, sem):
            return pltpu.make_async_remote_copy(
                src_ref=src, dst_ref=xg_ref.at[o_idx],
                send_sem=sem, recv_sem=recv_sems.at[o_idx],
                device_id=(to_pos,), device_id_type=pl.DeviceIdType.MESH)

        def wait_origin(o_idx):
            rdma = pltpu.make_async_remote_copy(
                src_ref=xg_ref.at[o_idx], dst_ref=xg_ref.at[o_idx],
                send_sem=cw_send_sems.at[0], recv_sem=recv_sems.at[o_idx],
                device_id=(left_pos,), device_id_type=pl.DeviceIdType.MESH)
            rdma.wait_recv()

        amax = jnp.float32(0.0)

        def gemm_half(src_block, row_base, i, acc):
            blk = jnp.dot(src_block, w_ref[...],
                          preferred_element_type=jnp.float32)
            out_ref[pl.ds(row_base + i * half, half), :] = blk
            return jnp.maximum(acc, jnp.max(jnp.abs(blk)))

        def stage_gemm(o_idx, acc):
            row_base = lut(o_idx, RING) * m_per
            cps = []
            for i in range(2):
                cp = pltpu.make_async_copy(
                    xg_ref.at[o_idx, pl.ds(i * half, half), :],
                    stage_ref.at[i], copy_sems.at[i])
                cp.start()
                cps.append(cp)
            for i in range(2):
                cps[i].wait()
                acc = gemm_half(stage_ref[i], row_base, i, acc)
            return acc

        o_cw0, o_ccw0 = m8(idx - 1), m8(idx + 1)
        o_cw1, o_ccw1 = m8(idx - 2), m8(idx + 2)
        o_c1 = m8(idx + 4)
        c1_src = m8(idx + 2 * par - 1)
        o_d3 = m8(idx - 3 + 6 * par)

        cw0 = mk_send(x_ref, idx, right_pos, cw_send_sems.at[0])
        ccw0 = mk_send(x_ref, idx, left_pos, ccw_send_sems.at[0])
        c0 = mk_send(x_ref, idx, p_pos, ch_send_sems.at[0])
        cw0.start()
        ccw0.start()
        c0.start()

        row_base = me_pos * m_per
        amax = gemm_half(x_ref[pl.ds(0, half), :], row_base, 0, amax)
        amax = gemm_half(x_ref[pl.ds(half, half), :], row_base, 1, amax)

        wait_origin(o_cw0)
        cw1 = mk_send(xg_ref.at[o_cw0], o_cw0, right_pos, cw_send_sems.at[1])
        cw1.start()
        wait_origin(o_ccw0)
        ccw1 = mk_send(xg_ref.at[o_ccw0], o_ccw0, left_pos,
                       ccw_send_sems.at[1])
        ccw1.start()
        c1 = mk_send(xg_ref.at[c1_src], c1_src, p_pos, ch_send_sems.at[1])
        c1.start()

        amax = stage_gemm(o_cw0, amax)
        amax = stage_gemm(o_ccw0, amax)
        wait_origin(p_idx)
        amax = stage_gemm(p_idx, amax)

        def mk_half(o_idx, i, to_pos, send_sem, recv_sem):
            sl = xg_ref.at[o_idx, pl.ds(i * half, half), :]
            return pltpu.make_async_remote_copy(
                src_ref=sl, dst_ref=sl, send_sem=send_sem, recv_sem=recv_sem,
                device_id=(to_pos,), device_id_type=pl.DeviceIdType.MESH)

        cw2a = mk_half(o_cw1, 0, right_pos, cw_send_sems.at[2],
                       recv_sems.at[o_cw1])
        ccw2a = mk_half(o_ccw1, 0, left_pos, ccw_send_sems.at[2],
                        recv_sems.at[o_ccw1])
        wait_origin(o_cw1)

        @pl.when(is_odd)
        def _():
            cw2a.start()

        wait_origin(o_ccw1)

        @pl.when(is_even)
        def _():
            ccw2a.start()

        amax = stage_gemm(o_cw1, amax)
        amax = stage_gemm(o_ccw1, amax)

        wait_origin(o_c1)
        d3b_cw = mk_half(o_c1, 1, right_pos, cw_send_sems.at[3],
                         d3b_sems.at[0])
        d3b_ccw = mk_half(o_c1, 1, left_pos, ccw_send_sems.at[3],
                          d3b_sems.at[0])

        @pl.when(is_even)
        def _():
            d3b_cw.start()

        @pl.when(is_odd)
        def _():
            d3b_ccw.start()

        amax = stage_gemm(o_c1, amax)

        row_d3 = lut(o_d3, RING) * m_per
        for i in range(2):
            sl = xg_ref.at[o_d3, pl.ds(i * half, half), :]
            rdma = pltpu.make_async_remote_copy(
                src_ref=sl, dst_ref=sl,
                send_sem=cw_send_sems.at[0],
                recv_sem=recv_sems.at[o_d3] if i == 0 else d3b_sems.at[0],
                device_id=(left_pos,), device_id_type=pl.DeviceIdType.MESH)
            rdma.wait_recv()
            cp = pltpu.make_async_copy(sl, stage_ref.at[i], copy_sems.at[i])
            cp.start()
            cp.wait()
            amax = gemm_half(stage_ref[i], row_d3, i, amax)

        for s in (cw0, cw1, ccw0, ccw1, c0, c1):
            s.wait_send()

        @pl.when(is_odd)
        def _():
            cw2a.wait_send()
            d3b_ccw.wait_send()

        @pl.when(is_even)
        def _():
            ccw2a.wait_send()
            d3b_cw.wait_send()

        amax_ref[pl.ds(me_pos, 1), :] = jnp.full((1, 128), amax, jnp.float32)
        a_rdmas = []
        for off in range(1, N_DEV):
            peer = m8(me_pos + off)
            rdma = pltpu.make_async_remote_copy(
                src_ref=amax_ref.at[pl.ds(me_pos, 1)],
                dst_ref=amax_ref.at[pl.ds(me_pos, 1)],
                send_sem=a_send_sems.at[off],
                recv_sem=a_recv_sems.at[me_pos],
                device_id=(peer,), device_id_type=pl.DeviceIdType.MESH)
            rdma.start()
            a_rdmas.append(rdma)
        for off in range(1, N_DEV):
            origin = m8(me_pos + off)
            recv = pltpu.make_async_remote_copy(
                src_ref=amax_ref.at[pl.ds(origin, 1)],
                dst_ref=amax_ref.at[pl.ds(origin, 1)],
                send_sem=a_send_sems.at[off],
                recv_sem=a_recv_sems.at[origin],
                device_id=(left_pos,), device_id_type=pl.DeviceIdType.MESH)
            recv.wait_recv()
        for rdma in a_rdmas:
            rdma.wait_send()

        gmax = jnp.max(amax_ref[...])
        scale = gmax / 127.0

        for j in range(N_DEV):
            rows = pl.ds(j * m_per, m_per)
            y = out_ref[rows, :]
            q = jnp.clip(jnp.round(y / scale), -127.0, 127.0)
            out_ref[rows, :] = q * scale

    out, _xg = pl.pallas_call(
        body,
        out_shape=[
            jax.ShapeDtypeStruct((N_DEV * m_per, n_per), jnp.float32),
            jax.ShapeDtypeStruct((N_DEV, m_per, k), jnp.float32),
        ],
        in_specs=[
            pl.BlockSpec(memory_space=pltpu.VMEM),
            pl.BlockSpec(memory_space=pltpu.VMEM),
        ],
        out_specs=[
            pl.BlockSpec(memory_space=pltpu.VMEM),
            pl.BlockSpec(memory_space=pl.ANY),
        ],
        scratch_shapes=[
            pltpu.VMEM((2, m_per // 2, k), jnp.float32),
            pltpu.VMEM((N_DEV, 128), jnp.float32),
            pltpu.SemaphoreType.DMA((4,)),
            pltpu.SemaphoreType.DMA((4,)),
            pltpu.SemaphoreType.DMA((2,)),
            pltpu.SemaphoreType.DMA((N_DEV,)),
            pltpu.SemaphoreType.DMA((1,)),
            pltpu.SemaphoreType.DMA((2,)),
            pltpu.SemaphoreType.DMA((N_DEV,)),
            pltpu.SemaphoreType.DMA((N_DEV,)),
        ],
        compiler_params=pltpu.CompilerParams(
            collective_id=0, vmem_limit_bytes=63 * 1024 * 1024),
    )(x, w_mat)
    return out


# device time: 279743 ns/iter; 2.4820x vs baseline; 1.0338x over previous
import jax
import jax.numpy as jnp
from jax import lax
from jax.experimental import pallas as pl
from jax.experimental.pallas import tpu as pltpu

N_DEV = 8
RING = [0, 4, 7, 3, 2, 6, 5, 1]
IDX = [0, 7, 4, 3, 1, 6, 5, 2]


def kernel(x, w_mat):
    m_per, k = x.shape
    _, n_per = w_mat.shape
    half = m_per // 2

    def body(x_ref, w_ref, out_ref, xg_ref, stage_ref, amax_ref,
             cw_send_sems, ccw_send_sems, ch_send_sems, recv_sems,
             d3b_sems, copy_sems, a_send_sems, a_recv_sems):
        me_pos = lax.axis_index("i")

        def lut(i, table):
            v = jnp.int32(table[0])
            for j in range(1, N_DEV):
                v = jnp.where(i == j, jnp.int32(table[j]), v)
            return v

        def m8(v):
            return lax.rem(v + 2 * N_DEV, N_DEV)

        idx = lut(me_pos, IDX)
        par = lax.rem(idx, 2)
        is_even = par == 0
        is_odd = par == 1

        left_pos = lut(m8(idx - 1), RING)
        right_pos = lut(m8(idx + 1), RING)
        p_idx = m8(idx + 3 - 6 * par)
        p_pos = lut(p_idx, RING)

        barrier_sem = pltpu.get_barrier_semaphore()
        for nbr in (left_pos, right_pos, p_pos):
            pl.semaphore_signal(
                barrier_sem, inc=1,
                device_id=(nbr,), device_id_type=pl.DeviceIdType.MESH,
            )
        pl.semaphore_wait(barrier_sem, 3)

        def mk_send(src, o_idx, to_pos, sem):
            return pltpu.make_async_remote_copy(
                src_ref=src, dst_ref=xg_ref.at[o_idx],
                send_sem=sem, recv_sem=recv_sems.at[o_idx],
                device_id=(to_pos,), device_id_type=pl.DeviceIdType.MESH)

        def wait_origin(o_idx):
            rdma = pltpu.make_async_remote_copy(
                src_ref=xg_ref.at[o_idx], dst_ref=xg_ref.at[o_idx],
                send_sem=cw_send_sems.at[0], recv_sem=recv_sems.at[o_idx],
                device_id=(left_pos,), device_id_type=pl.DeviceIdType.MESH)
            rdma.wait_recv()

        amax = jnp.float32(0.0)

        def gemm_half(src_block, row_base, i, acc):
            blk = jnp.dot(src_block, w_ref[...],
                          preferred_element_type=jnp.float32)
            out_ref[pl.ds(row_base + i * half, half), :] = blk
            return jnp.maximum(acc, jnp.max(jnp.abs(blk)))

        def stage_gemm(o_idx, acc):
            row_base = lut(o_idx, RING) * m_per
            cps = []
            for i in range(2):
                cp = pltpu.make_async_copy(
                    xg_ref.at[o_idx, pl.ds(i * half, half), :],
                    stage_ref.at[i], copy_sems.at[i])
                cp.start()
                cps.append(cp)
            for i in range(2):
                cps[i].wait()
                acc = gemm_half(stage_ref[i], row_base, i, acc)
            return acc

        o_cw0, o_ccw0 = m8(idx - 1), m8(idx + 1)
        o_cw1, o_ccw1 = m8(idx - 2), m8(idx + 2)
        o_c1 = m8(idx + 4)
        c1_src = m8(idx + 2 * par - 1)
        o_d3 = m8(idx - 3 + 6 * par)

        cw0 = mk_send(x_ref, idx, right_pos, cw_send_sems.at[0])
        ccw0 = mk_send(x_ref, idx, left_pos, ccw_send_sems.at[0])
        c0 = mk_send(x_ref, idx, p_pos, ch_send_sems.at[0])
        cw0.start()
        ccw0.start()
        c0.start()

        row_base = me_pos * m_per
        amax = gemm_half(x_ref[pl.ds(0, half), :], row_base, 0, amax)
        amax = gemm_half(x_ref[pl.ds(half, half), :], row_base, 1, amax)

        wait_origin(o_cw0)
        cw1 = mk_send(xg_ref.at[o_cw0], o_cw0, right_pos, cw_send_sems.at[1])
        cw1.start()
        wait_origin(o_ccw0)
        ccw1 = mk_send(xg_ref.at[o_ccw0], o_ccw0, left_pos,
                       ccw_send_sems.at[1])
        ccw1.start()
        c1 = mk_send(xg_ref.at[c1_src], c1_src, p_pos, ch_send_sems.at[1])
        c1.start()

        amax = stage_gemm(o_cw0, amax)
        amax = stage_gemm(o_ccw0, amax)
        wait_origin(p_idx)
        amax = stage_gemm(p_idx, amax)

        def mk_half(o_idx, i, to_pos, send_sem, recv_sem):
            sl = xg_ref.at[o_idx, pl.ds(i * half, half), :]
            return pltpu.make_async_remote_copy(
                src_ref=sl, dst_ref=sl, send_sem=send_sem, recv_sem=recv_sem,
                device_id=(to_pos,), device_id_type=pl.DeviceIdType.MESH)

        cw2a = mk_half(o_cw1, 0, right_pos, cw_send_sems.at[2],
                       recv_sems.at[o_cw1])
        ccw2a = mk_half(o_ccw1, 0, left_pos, ccw_send_sems.at[2],
                        recv_sems.at[o_ccw1])
        wait_origin(o_cw1)

        @pl.when(is_odd)
        def _():
            cw2a.start()

        wait_origin(o_ccw1)

        @pl.when(is_even)
        def _():
            ccw2a.start()

        wait_origin(o_c1)
        d3b_cw = mk_half(o_c1, 1, right_pos, cw_send_sems.at[3],
                         d3b_sems.at[0])
        d3b_ccw = mk_half(o_c1, 1, left_pos, ccw_send_sems.at[3],
                          d3b_sems.at[0])

        @pl.when(is_even)
        def _():
            d3b_cw.start()

        @pl.when(is_odd)
        def _():
            d3b_ccw.start()

        amax = stage_gemm(o_cw1, amax)
        amax = stage_gemm(o_ccw1, amax)
        amax = stage_gemm(o_c1, amax)

        row_d3 = lut(o_d3, RING) * m_per
        for i in range(2):
            sl = xg_ref.at[o_d3, pl.ds(i * half, half), :]
            rdma = pltpu.make_async_remote_copy(
                src_ref=sl, dst_ref=sl,
                send_sem=cw_send_sems.at[0],
                recv_sem=recv_sems.at[o_d3] if i == 0 else d3b_sems.at[0],
                device_id=(left_pos,), device_id_type=pl.DeviceIdType.MESH)
            rdma.wait_recv()
            cp = pltpu.make_async_copy(sl, stage_ref.at[i], copy_sems.at[i])
            cp.start()
            cp.wait()
            amax = gemm_half(stage_ref[i], row_d3, i, amax)

        for s in (cw0, cw1, ccw0, ccw1, c0, c1):
            s.wait_send()

        @pl.when(is_odd)
        def _():
            cw2a.wait_send()
            d3b_ccw.wait_send()

        @pl.when(is_even)
        def _():
            ccw2a.wait_send()
            d3b_cw.wait_send()

        amax_ref[pl.ds(me_pos, 1), :] = jnp.full((1, 128), amax, jnp.float32)
        a_rdmas = []
        for off in range(1, N_DEV):
            peer = m8(me_pos + off)
            rdma = pltpu.make_async_remote_copy(
                src_ref=amax_ref.at[pl.ds(me_pos, 1)],
                dst_ref=amax_ref.at[pl.ds(me_pos, 1)],
                send_sem=a_send_sems.at[off],
                recv_sem=a_recv_sems.at[me_pos],
                device_id=(peer,), device_id_type=pl.DeviceIdType.MESH)
            rdma.start()
            a_rdmas.append(rdma)
        for off in range(1, N_DEV):
            origin = m8(me_pos + off)
            recv = pltpu.make_async_remote_copy(
                src_ref=amax_ref.at[pl.ds(origin, 1)],
                dst_ref=amax_ref.at[pl.ds(origin, 1)],
                send_sem=a_send_sems.at[off],
                recv_sem=a_recv_sems.at[origin],
                device_id=(left_pos,), device_id_type=pl.DeviceIdType.MESH)
            recv.wait_recv()
        for rdma in a_rdmas:
            rdma.wait_send()

        gmax = jnp.max(amax_ref[...])
        scale = gmax / 127.0

        for j in range(N_DEV):
            rows = pl.ds(j * m_per, m_per)
            y = out_ref[rows, :]
            q = jnp.clip(jnp.round(y / scale), -127.0, 127.0)
            out_ref[rows, :] = q * scale

    out, _xg = pl.pallas_call(
        body,
        out_shape=[
            jax.ShapeDtypeStruct((N_DEV * m_per, n_per), jnp.float32),
            jax.ShapeDtypeStruct((N_DEV, m_per, k), jnp.float32),
        ],
        in_specs=[
            pl.BlockSpec(memory_space=pltpu.VMEM),
            pl.BlockSpec(memory_space=pltpu.VMEM),
        ],
        out_specs=[
            pl.BlockSpec(memory_space=pltpu.VMEM),
            pl.BlockSpec(memory_space=pl.ANY),
        ],
        scratch_shapes=[
            pltpu.VMEM((2, m_per // 2, k), jnp.float32),
            pltpu.VMEM((N_DEV, 128), jnp.float32),
            pltpu.SemaphoreType.DMA((4,)),
            pltpu.SemaphoreType.DMA((4,)),
            pltpu.SemaphoreType.DMA((2,)),
            pltpu.SemaphoreType.DMA((N_DEV,)),
            pltpu.SemaphoreType.DMA((1,)),
            pltpu.SemaphoreType.DMA((2,)),
            pltpu.SemaphoreType.DMA((N_DEV,)),
            pltpu.SemaphoreType.DMA((N_DEV,)),
        ],
        compiler_params=pltpu.CompilerParams(
            collective_id=0, vmem_limit_bytes=63 * 1024 * 1024),
    )(x, w_mat)
    return out
